# baseline (speedup 1.0000x reference)
import functools

import jax
import jax.numpy as jnp
from jax import lax
from jax.experimental import pallas as pl
from jax.experimental.pallas import tpu as pltpu
from jax.experimental.pallas import tpu_sc as plsc

_NC = 2
_NS = 16
_LN = 16
_DH = 16
_G = 128
_RG = 8
_CH = _G * _RG


@functools.lru_cache(maxsize=None)
def _make_side_kernel(n_pad, e_pad, n_splits):
    assert n_pad % (_NS * 8) == 0 and e_pad % (_NS * _CH) == 0
    assert n_splits % _NC == 0
    n_chunks = e_pad // (_NS * _CH)
    rows_per_tile = n_pad // _NS
    gps = e_pad // _G
    zr = 184
    assert rows_per_tile % zr == 0 and zr % 8 == 0
    mesh = plsc.VectorSubcoreMesh(core_axis_name="c", subcore_axis_name="s")

    @functools.partial(
        pl.kernel,
        out_type=jax.ShapeDtypeStruct((n_splits * n_pad, _DH), jnp.float32),
        mesh=mesh,
        compiler_params=pltpu.CompilerParams(use_tc_tiling_on_sc=False),
        scratch_types=[
            pltpu.VMEM((_RG, _G), jnp.int32),
            pltpu.VMEM((_RG, _G), jnp.int32),
            pltpu.VMEM((_CH, _DH), jnp.float32),
            pltpu.VMEM((_CH, _DH), jnp.float32),
            pltpu.VMEM((zr, _DH), jnp.float32),
            pltpu.VMEM_SHARED((n_pad, _DH), jnp.float32),
            pltpu.SemaphoreType.DMA,
        ],
    )
    def side_kernel(table, srcq, dstg, ab, out, idxv, dstv, av, rows, zbuf, acc, sem):
        c = lax.axis_index("c")
        s = lax.axis_index("s")
        base_row = s * rows_per_tile
        tile_g0 = s * (n_chunks * _RG)
        zeros16 = jnp.zeros((_LN,), jnp.float32)

        def zrow(i, carry):
            zbuf[i, pl.ds(0, _LN)] = zeros16
            return carry

        lax.fori_loop(0, zr, zrow, 0)

        for p in range(n_splits // _NC):
            sid = c + _NC * p

            for j in range(rows_per_tile // zr):
                pltpu.sync_copy(zbuf, acc.at[pl.ds(base_row + j * zr, zr)])
            plsc.subcore_barrier()

            def chunk(k, carry):
                g0 = tile_g0 + k * _RG
                e0 = g0 * _G
                d1 = pltpu.async_copy(srcq.at[pl.ds(sid * gps + g0, _RG)], idxv, sem)
                d2 = pltpu.async_copy(dstg.at[pl.ds(g0, _RG)], dstv, sem)
                d3 = pltpu.async_copy(ab.at[pl.ds(e0, _CH)], av, sem)
                d1.wait()

                descs = [
                    pltpu.async_copy(
                        table.at[idxv.at[g]], rows.at[pl.ds(g * _G, _G)], sem
                    )
                    for g in range(_RG)
                ]
                d2.wait()
                d3.wait()
                for dsc in descs:
                    dsc.wait()

                def scale(r, cc):
                    for j in range(_G):
                        e = r * _G + j
                        rows[e, pl.ds(0, _LN)] = (
                            rows[e, pl.ds(0, _LN)] * av[e, pl.ds(0, _LN)]
                        )
                    return cc

                lax.fori_loop(0, _RG, scale, 0)

                for g in range(_RG):
                    pltpu.sync_copy(
                        rows.at[pl.ds(g * _G, _G)], acc.at[dstv.at[g]], add=True
                    )
                return carry

            lax.fori_loop(0, n_chunks, chunk, 0)

            plsc.subcore_barrier()
            pltpu.sync_copy(
                acc.at[pl.ds(base_row, rows_per_tile)],
                out.at[pl.ds(sid * n_pad + base_row, rows_per_tile)],
            )

    return side_kernel


@functools.lru_cache(maxsize=None)
def _make_dense_kernel(n_nodes, d_in, d_out, br):
    assert n_nodes % br == 0 and d_in % _DH == 0
    n_parts = d_in // _DH

    def body(*refs):
        e_ref = refs[0]
        side_refs = refs[1:1 + n_parts]
        wg_ref, bg_ref, wb_ref, bb_ref = refs[1 + n_parts:5 + n_parts]
        e1_ref, n1_ref = refs[5 + n_parts:]
        e = e_ref[...]
        side = jnp.concatenate([r[...] for r in side_refs], axis=1)
        z = jnp.dot(e + side, wg_ref[...], preferred_element_type=jnp.float32)
        z = z + bg_ref[...]
        sum_emb = jnp.where(z >= 0, z, 0.01 * z)
        z2 = jnp.dot(e * side, wb_ref[...], preferred_element_type=jnp.float32)
        z2 = z2 + bb_ref[...]
        bi_emb = jnp.where(z2 >= 0, z2, 0.01 * z2)
        outv = sum_emb + bi_emb
        e1_ref[...] = outv
        nrm = jnp.sqrt(jnp.sum(outv * outv, axis=1, keepdims=True))
        n1_ref[...] = outv / jnp.maximum(nrm, 1e-12)

    return pl.pallas_call(
        body,
        grid=(n_nodes // br,),
        in_specs=[pl.BlockSpec((br, d_in), lambda i: (i, 0))]
        + [pl.BlockSpec((br, _DH), lambda i: (i, 0)) for _ in range(n_parts)]
        + [
            pl.BlockSpec((d_in, d_out), lambda i: (0, 0)),
            pl.BlockSpec((1, d_out), lambda i: (0, 0)),
            pl.BlockSpec((d_in, d_out), lambda i: (0, 0)),
            pl.BlockSpec((1, d_out), lambda i: (0, 0)),
        ],
        out_specs=[
            pl.BlockSpec((br, d_out), lambda i: (i, 0)),
            pl.BlockSpec((br, d_out), lambda i: (i, 0)),
        ],
        out_shape=[
            jax.ShapeDtypeStruct((n_nodes, d_out), jnp.float32),
            jax.ShapeDtypeStruct((n_nodes, d_out), jnp.float32),
        ],
    )


def kernel(ego_embeddings, edge_index, a_values,
           W_gc0, b_gc0, W_bi0, b_bi0,
           W_gc1, b_gc1, W_bi1, b_bi1):
    n, emb = ego_embeddings.shape
    n_edges = a_values.shape[0]
    h1 = W_gc0.shape[1]
    h2 = W_gc1.shape[1]

    q = _NS * _CH
    e_pad = ((n_edges + q - 1) // q) * q
    pad = e_pad - n_edges
    src = edge_index[0]
    dst = edge_index[1]
    a = a_values
    if pad:
        pad_idx = (jnp.arange(pad, dtype=jnp.int32) * 97) % n
        src = jnp.concatenate([src, pad_idx])
        dst = jnp.concatenate([dst, pad_idx])
        a = jnp.concatenate([a, jnp.zeros((pad,), jnp.float32)])
    dstg = dst.reshape(e_pad // _G, _G)
    ab = jnp.broadcast_to(a[:, None], (e_pad, _DH))

    def _slice_idx(ns):
        q = jnp.arange(ns, dtype=jnp.int32)[:, None]
        return (src[None, :] * ns + q).reshape(ns * (e_pad // _G), _G)

    n_pad = ((n + _NS * 8 - 1) // (_NS * 8)) * (_NS * 8)

    ns0 = emb // _DH
    table0 = ego_embeddings.reshape(ns0 * n, _DH)
    acc0 = _make_side_kernel(n_pad, e_pad, ns0)(table0, _slice_idx(ns0), dstg, ab)
    sides0 = [acc0[i * n_pad:i * n_pad + n] for i in range(ns0)]
    e1, n1 = _make_dense_kernel(n, emb, h1, 2000)(
        ego_embeddings, *sides0, W_gc0, b_gc0, W_bi0, b_bi0
    )

    ns1 = h1 // _DH
    table1 = e1.reshape(ns1 * n, _DH)
    acc1 = _make_side_kernel(n_pad, e_pad, ns1)(table1, _slice_idx(ns1), dstg, ab)
    sides1 = [acc1[i * n_pad:i * n_pad + n] for i in range(ns1)]
    _, n2 = _make_dense_kernel(n, h1, h2, 2000)(
        e1, *sides1, W_gc1, b_gc1, W_bi1, b_bi1
    )

    return jnp.concatenate([ego_embeddings, n1, n2], axis=1)


# speedup vs baseline: 1.5551x; 1.5551x over previous
import functools

import jax
import jax.numpy as jnp
from jax import lax
from jax.experimental import pallas as pl
from jax.experimental.pallas import tpu as pltpu
from jax.experimental.pallas import tpu_sc as plsc

_NC = 2
_NS = 16
_LN = 16
_DH = 16
_G = 128
_RG = 8
_CH = _G * _RG


def _bcast(vec, lane):
    s = lax.squeeze(lax.slice(vec, (lane,), (lane + 1,)), (0,))
    return lax.broadcast_in_dim(s, (_LN,), ())


@functools.lru_cache(maxsize=None)
def _make_side_kernel(n_pad, e_pad, n_splits):
    assert n_pad % (_NS * 8) == 0 and e_pad % (_NS * _CH) == 0
    assert n_splits % _NC == 0
    n_chunks = e_pad // (_NS * _CH)
    rows_per_tile = n_pad // _NS
    gps = e_pad // _G
    zr = 184
    assert rows_per_tile % zr == 0 and zr % 8 == 0
    mesh = plsc.VectorSubcoreMesh(core_axis_name="c", subcore_axis_name="s")

    @functools.partial(
        pl.kernel,
        out_type=jax.ShapeDtypeStruct((n_splits * n_pad, _DH), jnp.float32),
        mesh=mesh,
        compiler_params=pltpu.CompilerParams(use_tc_tiling_on_sc=False),
        scratch_types=[
            pltpu.VMEM((_RG, _G), jnp.int32),
            pltpu.VMEM((_RG, _G), jnp.int32),
            pltpu.VMEM((_RG, _G), jnp.float32),
            pltpu.VMEM((_CH, _DH), jnp.float32),
            pltpu.VMEM((zr, _DH), jnp.float32),
            pltpu.VMEM_SHARED((n_pad, _DH), jnp.float32),
            pltpu.SemaphoreType.DMA,
        ],
    )
    def side_kernel(table, srcq, dstg, ag, out, idxv, dstv, av, rows, zbuf, acc, sem):
        c = lax.axis_index("c")
        s = lax.axis_index("s")
        base_row = s * rows_per_tile
        tile_g0 = s * (n_chunks * _RG)
        zeros16 = jnp.zeros((_LN,), jnp.float32)

        def zrow(i, carry):
            zbuf[i, pl.ds(0, _LN)] = zeros16
            return carry

        lax.fori_loop(0, zr, zrow, 0)

        for p in range(n_splits // _NC):
            sid = c + _NC * p

            for j in range(rows_per_tile // zr):
                pltpu.sync_copy(zbuf, acc.at[pl.ds(base_row + j * zr, zr)])
            plsc.subcore_barrier()

            def chunk(k, carry):
                g0 = tile_g0 + k * _RG
                d1 = pltpu.async_copy(srcq.at[pl.ds(sid * gps + g0, _RG)], idxv, sem)
                d2 = pltpu.async_copy(dstg.at[pl.ds(g0, _RG)], dstv, sem)
                d3 = pltpu.async_copy(ag.at[pl.ds(g0, _RG)], av, sem)
                d1.wait()

                descs = [
                    pltpu.async_copy(
                        table.at[idxv.at[g]], rows.at[pl.ds(g * _G, _G)], sem
                    )
                    for g in range(_RG)
                ]
                d2.wait()
                d3.wait()
                for dsc in descs:
                    dsc.wait()

                def scale(r, cc):
                    for l in range(_G // _LN):
                        gain = av[r, pl.ds(l * _LN, _LN)]
                        for lane in range(_LN):
                            e = r * _G + l * _LN + lane
                            b = _bcast(gain, lane)
                            rows[e, pl.ds(0, _LN)] = rows[e, pl.ds(0, _LN)] * b
                    return cc

                lax.fori_loop(0, _RG, scale, 0)

                for g in range(_RG):
                    pltpu.sync_copy(
                        rows.at[pl.ds(g * _G, _G)], acc.at[dstv.at[g]], add=True
                    )
                return carry

            lax.fori_loop(0, n_chunks, chunk, 0)

            plsc.subcore_barrier()
            pltpu.sync_copy(
                acc.at[pl.ds(base_row, rows_per_tile)],
                out.at[pl.ds(sid * n_pad + base_row, rows_per_tile)],
            )

    return side_kernel


@functools.lru_cache(maxsize=None)
def _make_dense_kernel(n_nodes, d_in, d_out, br):
    assert n_nodes % br == 0 and d_in % _DH == 0
    n_parts = d_in // _DH

    def body(*refs):
        e_ref = refs[0]
        side_refs = refs[1:1 + n_parts]
        wg_ref, bg_ref, wb_ref, bb_ref = refs[1 + n_parts:5 + n_parts]
        e1_ref, n1_ref = refs[5 + n_parts:]
        e = e_ref[...]
        side = jnp.concatenate([r[...] for r in side_refs], axis=1)
        z = jnp.dot(e + side, wg_ref[...], preferred_element_type=jnp.float32)
        z = z + bg_ref[...]
        sum_emb = jnp.where(z >= 0, z, 0.01 * z)
        z2 = jnp.dot(e * side, wb_ref[...], preferred_element_type=jnp.float32)
        z2 = z2 + bb_ref[...]
        bi_emb = jnp.where(z2 >= 0, z2, 0.01 * z2)
        outv = sum_emb + bi_emb
        e1_ref[...] = outv
        nrm = jnp.sqrt(jnp.sum(outv * outv, axis=1, keepdims=True))
        n1_ref[...] = outv / jnp.maximum(nrm, 1e-12)

    return pl.pallas_call(
        body,
        grid=(n_nodes // br,),
        in_specs=[pl.BlockSpec((br, d_in), lambda i: (i, 0))]
        + [pl.BlockSpec((br, _DH), lambda i: (i, 0)) for _ in range(n_parts)]
        + [
            pl.BlockSpec((d_in, d_out), lambda i: (0, 0)),
            pl.BlockSpec((1, d_out), lambda i: (0, 0)),
            pl.BlockSpec((d_in, d_out), lambda i: (0, 0)),
            pl.BlockSpec((1, d_out), lambda i: (0, 0)),
        ],
        out_specs=[
            pl.BlockSpec((br, d_out), lambda i: (i, 0)),
            pl.BlockSpec((br, d_out), lambda i: (i, 0)),
        ],
        out_shape=[
            jax.ShapeDtypeStruct((n_nodes, d_out), jnp.float32),
            jax.ShapeDtypeStruct((n_nodes, d_out), jnp.float32),
        ],
    )


def kernel(ego_embeddings, edge_index, a_values,
           W_gc0, b_gc0, W_bi0, b_bi0,
           W_gc1, b_gc1, W_bi1, b_bi1):
    n, emb = ego_embeddings.shape
    n_edges = a_values.shape[0]
    h1 = W_gc0.shape[1]
    h2 = W_gc1.shape[1]

    q = _NS * _CH
    e_pad = ((n_edges + q - 1) // q) * q
    pad = e_pad - n_edges
    src = edge_index[0]
    dst = edge_index[1]
    a = a_values
    if pad:
        pad_idx = (jnp.arange(pad, dtype=jnp.int32) * 97) % n
        src = jnp.concatenate([src, pad_idx])
        dst = jnp.concatenate([dst, pad_idx])
        a = jnp.concatenate([a, jnp.zeros((pad,), jnp.float32)])
    dstg = dst.reshape(e_pad // _G, _G)
    ag = a.reshape(e_pad // _G, _G)

    def _slice_idx(ns):
        q = jnp.arange(ns, dtype=jnp.int32)[:, None]
        return (src[None, :] * ns + q).reshape(ns * (e_pad // _G), _G)

    n_pad = ((n + _NS * 8 - 1) // (_NS * 8)) * (_NS * 8)

    ns0 = emb // _DH
    table0 = ego_embeddings.reshape(ns0 * n, _DH)
    acc0 = _make_side_kernel(n_pad, e_pad, ns0)(table0, _slice_idx(ns0), dstg, ag)
    sides0 = [acc0[i * n_pad:i * n_pad + n] for i in range(ns0)]
    e1, n1 = _make_dense_kernel(n, emb, h1, 2000)(
        ego_embeddings, *sides0, W_gc0, b_gc0, W_bi0, b_bi0
    )

    ns1 = h1 // _DH
    table1 = e1.reshape(ns1 * n, _DH)
    acc1 = _make_side_kernel(n_pad, e_pad, ns1)(table1, _slice_idx(ns1), dstg, ag)
    sides1 = [acc1[i * n_pad:i * n_pad + n] for i in range(ns1)]
    _, n2 = _make_dense_kernel(n, h1, h2, 2000)(
        e1, *sides1, W_gc1, b_gc1, W_bi1, b_bi1
    )

    return jnp.concatenate([ego_embeddings, n1, n2], axis=1)


# speedup vs baseline: 1.6505x; 1.0613x over previous
import functools

import jax
import jax.numpy as jnp
from jax import lax
from jax.experimental import pallas as pl
from jax.experimental.pallas import tpu as pltpu
from jax.experimental.pallas import tpu_sc as plsc

_NC = 2
_NS = 16
_LN = 16
_DH = 16
_G = 128
_RG = 8
_CH = _G * _RG


def _bcast(vec, lane):
    s = lax.squeeze(lax.slice(vec, (lane,), (lane + 1,)), (0,))
    return lax.broadcast_in_dim(s, (_LN,), ())


@functools.lru_cache(maxsize=None)
def _make_side_kernel(n_pad, e_pad, n_splits):
    assert n_pad % (_NS * 8) == 0 and e_pad % (_NS * _CH * 2) == 0
    assert n_splits % _NC == 0
    n_chunks = e_pad // (_NS * _CH)
    n_pairs = n_chunks // 2
    rows_per_tile = n_pad // _NS
    gps = e_pad // _G
    zr = 184
    assert rows_per_tile % zr == 0 and zr % 8 == 0
    mesh = plsc.VectorSubcoreMesh(core_axis_name="c", subcore_axis_name="s")

    @functools.partial(
        pl.kernel,
        out_type=jax.ShapeDtypeStruct((n_splits * n_pad, _DH), jnp.float32),
        mesh=mesh,
        compiler_params=pltpu.CompilerParams(use_tc_tiling_on_sc=False),
        scratch_types=[
            pltpu.VMEM((2, _RG, _G), jnp.int32),
            pltpu.VMEM((2, _RG, _G), jnp.int32),
            pltpu.VMEM((2, _RG, _G), jnp.float32),
            pltpu.VMEM((2, _CH, _DH), jnp.float32),
            pltpu.VMEM((zr, _DH), jnp.float32),
            pltpu.VMEM_SHARED((n_pad, _DH), jnp.float32),
            pltpu.SemaphoreType.DMA,
            pltpu.SemaphoreType.DMA,
        ],
    )
    def side_kernel(table, srcq, dstg, ag, out, idxv, dstv, av, rows,
                    zbuf, acc, semA, semB):
        c = lax.axis_index("c")
        s = lax.axis_index("s")
        base_row = s * rows_per_tile
        tile_g0 = s * (n_chunks * _RG)
        zeros16 = jnp.zeros((_LN,), jnp.float32)
        sems = (semA, semB)

        def zrow(i, carry):
            zbuf[i, pl.ds(0, _LN)] = zeros16
            return carry

        lax.fori_loop(0, zr, zrow, 0)

        for p in range(n_splits // _NC):
            sid = c + _NC * p

            for j in range(rows_per_tile // zr):
                pltpu.sync_copy(zbuf, acc.at[pl.ds(base_row + j * zr, zr)])
            plsc.subcore_barrier()

            def prefetch(b, k):
                g0 = tile_g0 + k * _RG
                pltpu.sync_copy(srcq.at[pl.ds(sid * gps + g0, _RG)], idxv.at[b])
                pltpu.sync_copy(dstg.at[pl.ds(g0, _RG)], dstv.at[b])
                pltpu.sync_copy(ag.at[pl.ds(g0, _RG)], av.at[b])
                for g in range(_RG):
                    pltpu.async_copy(
                        table.at[idxv.at[b].at[g]],
                        rows.at[b].at[pl.ds(g * _G, _G)],
                        sems[b],
                    )

            def consume(b):
                for g in range(_RG):
                    pltpu.make_async_copy(
                        table.at[idxv.at[b].at[g]],
                        rows.at[b].at[pl.ds(g * _G, _G)],
                        sems[b],
                    ).wait()

                def scale(r, cc):
                    for l in range(_G // _LN):
                        gain = av[b, r, pl.ds(l * _LN, _LN)]
                        for lane in range(_LN):
                            e = r * _G + l * _LN + lane
                            bc = _bcast(gain, lane)
                            rows[b, e, pl.ds(0, _LN)] = (
                                rows[b, e, pl.ds(0, _LN)] * bc
                            )
                    return cc

                lax.fori_loop(0, _RG, scale, 0)

                for g in range(_RG):
                    pltpu.sync_copy(
                        rows.at[b].at[pl.ds(g * _G, _G)],
                        acc.at[dstv.at[b].at[g]],
                        add=True,
                    )

            prefetch(0, 0)

            def pair(j, carry):
                prefetch(1, 2 * j + 1)
                consume(0)
                prefetch(0, 2 * j + 2)
                consume(1)
                return carry

            lax.fori_loop(0, n_pairs - 1, pair, 0)
            prefetch(1, n_chunks - 1)
            consume(0)
            consume(1)

            plsc.subcore_barrier()
            pltpu.sync_copy(
                acc.at[pl.ds(base_row, rows_per_tile)],
                out.at[pl.ds(sid * n_pad + base_row, rows_per_tile)],
            )

    return side_kernel


@functools.lru_cache(maxsize=None)
def _make_dense_kernel(n_nodes, d_in, d_out, br):
    assert n_nodes % br == 0 and d_in % _DH == 0
    n_parts = d_in // _DH

    def body(*refs):
        e_ref = refs[0]
        side_refs = refs[1:1 + n_parts]
        wg_ref, bg_ref, wb_ref, bb_ref = refs[1 + n_parts:5 + n_parts]
        e1_ref, n1_ref = refs[5 + n_parts:]
        e = e_ref[...]
        side = jnp.concatenate([r[...] for r in side_refs], axis=1)
        z = jnp.dot(e + side, wg_ref[...], preferred_element_type=jnp.float32)
        z = z + bg_ref[...]
        sum_emb = jnp.where(z >= 0, z, 0.01 * z)
        z2 = jnp.dot(e * side, wb_ref[...], preferred_element_type=jnp.float32)
        z2 = z2 + bb_ref[...]
        bi_emb = jnp.where(z2 >= 0, z2, 0.01 * z2)
        outv = sum_emb + bi_emb
        e1_ref[...] = outv
        nrm = jnp.sqrt(jnp.sum(outv * outv, axis=1, keepdims=True))
        n1_ref[...] = outv / jnp.maximum(nrm, 1e-12)

    return pl.pallas_call(
        body,
        grid=(n_nodes // br,),
        in_specs=[pl.BlockSpec((br, d_in), lambda i: (i, 0))]
        + [pl.BlockSpec((br, _DH), lambda i: (i, 0)) for _ in range(n_parts)]
        + [
            pl.BlockSpec((d_in, d_out), lambda i: (0, 0)),
            pl.BlockSpec((1, d_out), lambda i: (0, 0)),
            pl.BlockSpec((d_in, d_out), lambda i: (0, 0)),
            pl.BlockSpec((1, d_out), lambda i: (0, 0)),
        ],
        out_specs=[
            pl.BlockSpec((br, d_out), lambda i: (i, 0)),
            pl.BlockSpec((br, d_out), lambda i: (i, 0)),
        ],
        out_shape=[
            jax.ShapeDtypeStruct((n_nodes, d_out), jnp.float32),
            jax.ShapeDtypeStruct((n_nodes, d_out), jnp.float32),
        ],
    )


def kernel(ego_embeddings, edge_index, a_values,
           W_gc0, b_gc0, W_bi0, b_bi0,
           W_gc1, b_gc1, W_bi1, b_bi1):
    n, emb = ego_embeddings.shape
    n_edges = a_values.shape[0]
    h1 = W_gc0.shape[1]
    h2 = W_gc1.shape[1]

    q = _NS * _CH * 2
    e_pad = ((n_edges + q - 1) // q) * q
    pad = e_pad - n_edges
    src = edge_index[0]
    dst = edge_index[1]
    a = a_values
    if pad:
        pad_idx = (jnp.arange(pad, dtype=jnp.int32) * 97) % n
        src = jnp.concatenate([src, pad_idx])
        dst = jnp.concatenate([dst, pad_idx])
        a = jnp.concatenate([a, jnp.zeros((pad,), jnp.float32)])
    dstg = dst.reshape(e_pad // _G, _G)
    ag = a.reshape(e_pad // _G, _G)

    def _slice_idx(ns):
        q = jnp.arange(ns, dtype=jnp.int32)[:, None]
        return (src[None, :] * ns + q).reshape(ns * (e_pad // _G), _G)

    n_pad = ((n + _NS * 8 - 1) // (_NS * 8)) * (_NS * 8)

    ns0 = emb // _DH
    table0 = ego_embeddings.reshape(ns0 * n, _DH)
    acc0 = _make_side_kernel(n_pad, e_pad, ns0)(table0, _slice_idx(ns0), dstg, ag)
    sides0 = [acc0[i * n_pad:i * n_pad + n] for i in range(ns0)]
    e1, n1 = _make_dense_kernel(n, emb, h1, 2000)(
        ego_embeddings, *sides0, W_gc0, b_gc0, W_bi0, b_bi0
    )

    ns1 = h1 // _DH
    table1 = e1.reshape(ns1 * n, _DH)
    acc1 = _make_side_kernel(n_pad, e_pad, ns1)(table1, _slice_idx(ns1), dstg, ag)
    sides1 = [acc1[i * n_pad:i * n_pad + n] for i in range(ns1)]
    _, n2 = _make_dense_kernel(n, h1, h2, 2000)(
        e1, *sides1, W_gc1, b_gc1, W_bi1, b_bi1
    )

    return jnp.concatenate([ego_embeddings, n1, n2], axis=1)


# speedup vs baseline: 1.9584x; 1.1865x over previous
import functools

import jax
import jax.numpy as jnp
from jax import lax
from jax.experimental import pallas as pl
from jax.experimental.pallas import tpu as pltpu
from jax.experimental.pallas import tpu_sc as plsc

_NC = 2
_NS = 16
_LN = 16
_DH = 16
_G = 128
_RG = 8
_CH = _G * _RG


def _bcast(vec, lane):
    s = lax.squeeze(lax.slice(vec, (lane,), (lane + 1,)), (0,))
    return lax.broadcast_in_dim(s, (_LN,), ())


@functools.lru_cache(maxsize=None)
def _make_side_kernel(n_pad, e_pad, n_splits):
    assert n_pad % (_NS * 8) == 0 and e_pad % (_NS * _CH * 2) == 0
    assert n_splits % _NC == 0
    n_chunks = e_pad // (_NS * _CH)
    n_pairs = n_chunks // 2
    rows_per_tile = n_pad // _NS
    gps = e_pad // _G
    zr = 184
    assert rows_per_tile % zr == 0 and zr % 8 == 0
    mesh = plsc.VectorSubcoreMesh(core_axis_name="c", subcore_axis_name="s")

    @functools.partial(
        pl.kernel,
        out_type=jax.ShapeDtypeStruct((n_splits * n_pad, _DH), jnp.float32),
        mesh=mesh,
        compiler_params=pltpu.CompilerParams(use_tc_tiling_on_sc=False),
        scratch_types=[
            pltpu.VMEM((2, _RG, _G), jnp.int32),
            pltpu.VMEM((2, _RG, _G), jnp.int32),
            pltpu.VMEM((2, _RG, _G), jnp.float32),
            pltpu.VMEM((2, _CH, _DH), jnp.float32),
            pltpu.VMEM((zr, _DH), jnp.float32),
            pltpu.VMEM_SHARED((n_pad, _DH), jnp.float32),
            pltpu.SemaphoreType.DMA,
            pltpu.SemaphoreType.DMA,
            pltpu.SemaphoreType.DMA,
            pltpu.SemaphoreType.DMA,
        ],
    )
    def side_kernel(table, srcq, dstg, ag, out, idxv, dstv, av, rows,
                    zbuf, acc, semA, semB, semC, semD):
        c = lax.axis_index("c")
        s = lax.axis_index("s")
        base_row = s * rows_per_tile
        tile_g0 = s * (n_chunks * _RG)
        zeros16 = jnp.zeros((_LN,), jnp.float32)
        sems = (semA, semB)
        ssems = (semC, semD)

        def zrow(i, carry):
            zbuf[i, pl.ds(0, _LN)] = zeros16
            return carry

        lax.fori_loop(0, zr, zrow, 0)

        for p in range(n_splits // _NC):
            sid = c + _NC * p

            for j in range(rows_per_tile // zr):
                pltpu.sync_copy(zbuf, acc.at[pl.ds(base_row + j * zr, zr)])
            plsc.subcore_barrier()

            def prefetch(b, k):
                g0 = tile_g0 + k * _RG
                pltpu.sync_copy(srcq.at[pl.ds(sid * gps + g0, _RG)], idxv.at[b])
                pltpu.sync_copy(dstg.at[pl.ds(g0, _RG)], dstv.at[b])
                pltpu.sync_copy(ag.at[pl.ds(g0, _RG)], av.at[b])
                for g in range(_RG):
                    pltpu.async_copy(
                        table.at[idxv.at[b].at[g]],
                        rows.at[b].at[pl.ds(g * _G, _G)],
                        sems[b],
                    )

            def consume(b):
                for g in range(_RG):
                    pltpu.make_async_copy(
                        table.at[idxv.at[b].at[g]],
                        rows.at[b].at[pl.ds(g * _G, _G)],
                        sems[b],
                    ).wait()

                    def scale(l, cc):
                        gain = av[b, g, pl.ds(l * _LN, _LN)]
                        for lane in range(_LN):
                            e = g * _G + l * _LN + lane
                            bc = _bcast(gain, lane)
                            rows[b, e, pl.ds(0, _LN)] = (
                                rows[b, e, pl.ds(0, _LN)] * bc
                            )
                        return cc

                    lax.fori_loop(0, _G // _LN, scale, 0)

                    pltpu.async_copy(
                        rows.at[b].at[pl.ds(g * _G, _G)],
                        acc.at[dstv.at[b].at[g]],
                        ssems[b],
                        add=True,
                    )

                for g in range(_RG):
                    pltpu.make_async_copy(
                        rows.at[b].at[pl.ds(g * _G, _G)],
                        acc.at[dstv.at[b].at[g]],
                        ssems[b],
                    ).wait()

            prefetch(0, 0)

            def pair(j, carry):
                prefetch(1, 2 * j + 1)
                consume(0)
                prefetch(0, 2 * j + 2)
                consume(1)
                return carry

            lax.fori_loop(0, n_pairs - 1, pair, 0)
            prefetch(1, n_chunks - 1)
            consume(0)
            consume(1)

            plsc.subcore_barrier()
            pltpu.sync_copy(
                acc.at[pl.ds(base_row, rows_per_tile)],
                out.at[pl.ds(sid * n_pad + base_row, rows_per_tile)],
            )

    return side_kernel


@functools.lru_cache(maxsize=None)
def _make_dense_kernel(n_nodes, d_in, d_out, br):
    assert n_nodes % br == 0 and d_in % _DH == 0
    n_parts = d_in // _DH

    def body(*refs):
        e_ref = refs[0]
        side_refs = refs[1:1 + n_parts]
        wg_ref, bg_ref, wb_ref, bb_ref = refs[1 + n_parts:5 + n_parts]
        e1_ref, n1_ref = refs[5 + n_parts:]
        e = e_ref[...]
        side = jnp.concatenate([r[...] for r in side_refs], axis=1)
        z = jnp.dot(e + side, wg_ref[...], preferred_element_type=jnp.float32)
        z = z + bg_ref[...]
        sum_emb = jnp.where(z >= 0, z, 0.01 * z)
        z2 = jnp.dot(e * side, wb_ref[...], preferred_element_type=jnp.float32)
        z2 = z2 + bb_ref[...]
        bi_emb = jnp.where(z2 >= 0, z2, 0.01 * z2)
        outv = sum_emb + bi_emb
        e1_ref[...] = outv
        nrm = jnp.sqrt(jnp.sum(outv * outv, axis=1, keepdims=True))
        n1_ref[...] = outv / jnp.maximum(nrm, 1e-12)

    return pl.pallas_call(
        body,
        grid=(n_nodes // br,),
        in_specs=[pl.BlockSpec((br, d_in), lambda i: (i, 0))]
        + [pl.BlockSpec((br, _DH), lambda i: (i, 0)) for _ in range(n_parts)]
        + [
            pl.BlockSpec((d_in, d_out), lambda i: (0, 0)),
            pl.BlockSpec((1, d_out), lambda i: (0, 0)),
            pl.BlockSpec((d_in, d_out), lambda i: (0, 0)),
            pl.BlockSpec((1, d_out), lambda i: (0, 0)),
        ],
        out_specs=[
            pl.BlockSpec((br, d_out), lambda i: (i, 0)),
            pl.BlockSpec((br, d_out), lambda i: (i, 0)),
        ],
        out_shape=[
            jax.ShapeDtypeStruct((n_nodes, d_out), jnp.float32),
            jax.ShapeDtypeStruct((n_nodes, d_out), jnp.float32),
        ],
    )


def kernel(ego_embeddings, edge_index, a_values,
           W_gc0, b_gc0, W_bi0, b_bi0,
           W_gc1, b_gc1, W_bi1, b_bi1):
    n, emb = ego_embeddings.shape
    n_edges = a_values.shape[0]
    h1 = W_gc0.shape[1]
    h2 = W_gc1.shape[1]

    q = _NS * _CH * 2
    e_pad = ((n_edges + q - 1) // q) * q
    pad = e_pad - n_edges
    src = edge_index[0]
    dst = edge_index[1]
    a = a_values
    if pad:
        pad_idx = (jnp.arange(pad, dtype=jnp.int32) * 97) % n
        src = jnp.concatenate([src, pad_idx])
        dst = jnp.concatenate([dst, pad_idx])
        a = jnp.concatenate([a, jnp.zeros((pad,), jnp.float32)])
    dstg = dst.reshape(e_pad // _G, _G)
    ag = a.reshape(e_pad // _G, _G)

    def _slice_idx(ns):
        q = jnp.arange(ns, dtype=jnp.int32)[:, None]
        return (src[None, :] * ns + q).reshape(ns * (e_pad // _G), _G)

    n_pad = ((n + _NS * 8 - 1) // (_NS * 8)) * (_NS * 8)

    ns0 = emb // _DH
    table0 = ego_embeddings.reshape(ns0 * n, _DH)
    acc0 = _make_side_kernel(n_pad, e_pad, ns0)(table0, _slice_idx(ns0), dstg, ag)
    sides0 = [acc0[i * n_pad:i * n_pad + n] for i in range(ns0)]
    e1, n1 = _make_dense_kernel(n, emb, h1, 2000)(
        ego_embeddings, *sides0, W_gc0, b_gc0, W_bi0, b_bi0
    )

    ns1 = h1 // _DH
    table1 = e1.reshape(ns1 * n, _DH)
    acc1 = _make_side_kernel(n_pad, e_pad, ns1)(table1, _slice_idx(ns1), dstg, ag)
    sides1 = [acc1[i * n_pad:i * n_pad + n] for i in range(ns1)]
    _, n2 = _make_dense_kernel(n, h1, h2, 2000)(
        e1, *sides1, W_gc1, b_gc1, W_bi1, b_bi1
    )

    return jnp.concatenate([ego_embeddings, n1, n2], axis=1)


# speedup vs baseline: 2.3253x; 1.1874x over previous
import functools

import jax
import jax.numpy as jnp
from jax import lax
from jax.experimental import pallas as pl
from jax.experimental.pallas import tpu as pltpu
from jax.experimental.pallas import tpu_sc as plsc

_NC = 2
_NS = 16
_LN = 16
_DH = 16
_G = 128
_RG = 8
_CH = _G * _RG


def _bcast(vec, lane):
    s = lax.squeeze(lax.slice(vec, (lane,), (lane + 1,)), (0,))
    return lax.broadcast_in_dim(s, (_LN,), ())


@functools.lru_cache(maxsize=None)
def _make_side_kernel(n_pad, e_pad, n_splits):
    assert n_pad % (_NS * 8) == 0 and e_pad % (_NS * _CH * 2) == 0
    assert n_splits % _NC == 0
    n_chunks = e_pad // (_NS * _CH)
    n_pairs = n_chunks // 2
    rows_per_tile = n_pad // _NS
    zr = 184
    assert rows_per_tile % zr == 0 and zr % 8 == 0
    mesh = plsc.VectorSubcoreMesh(core_axis_name="c", subcore_axis_name="s")

    @functools.partial(
        pl.kernel,
        out_type=jax.ShapeDtypeStruct((n_pad, n_splits * _DH), jnp.float32),
        mesh=mesh,
        compiler_params=pltpu.CompilerParams(use_tc_tiling_on_sc=False),
        scratch_types=[
            pltpu.VMEM((2, _RG, _G), jnp.int32),
            pltpu.VMEM((2, _RG, _G), jnp.int32),
            pltpu.VMEM((2, _RG, _G), jnp.float32),
            pltpu.VMEM((2, _CH, _DH), jnp.float32),
            pltpu.VMEM((zr, _DH), jnp.float32),
            pltpu.VMEM_SHARED((n_pad, _DH), jnp.float32),
            pltpu.SemaphoreType.DMA,
            pltpu.SemaphoreType.DMA,
            pltpu.SemaphoreType.DMA,
            pltpu.SemaphoreType.DMA,
        ],
    )
    def side_kernel(table, srcg, dstg, ag, out, idxv, dstv, av, rows,
                    zbuf, acc, semA, semB, semC, semD):
        c = lax.axis_index("c")
        s = lax.axis_index("s")
        base_row = s * rows_per_tile
        tile_g0 = s * (n_chunks * _RG)
        zeros16 = jnp.zeros((_LN,), jnp.float32)
        sems = (semA, semB)
        ssems = (semC, semD)

        def zrow(i, carry):
            zbuf[i, pl.ds(0, _LN)] = zeros16
            return carry

        lax.fori_loop(0, zr, zrow, 0)

        for p in range(n_splits // _NC):
            sid = c + _NC * p
            vsid = lax.broadcast_in_dim(sid, (_LN,), ())

            for j in range(rows_per_tile // zr):
                pltpu.sync_copy(zbuf, acc.at[pl.ds(base_row + j * zr, zr)])
            plsc.subcore_barrier()

            def prefetch(b, k):
                g0 = tile_g0 + k * _RG
                pltpu.sync_copy(srcg.at[pl.ds(g0, _RG)], idxv.at[b])
                pltpu.sync_copy(dstg.at[pl.ds(g0, _RG)], dstv.at[b])
                pltpu.sync_copy(ag.at[pl.ds(g0, _RG)], av.at[b])
                for g in range(_RG):
                    def xform(l, cc):
                        sl = idxv[b, g, pl.ds(l * _LN, _LN)]
                        idxv[b, g, pl.ds(l * _LN, _LN)] = (
                            sl * n_splits + vsid
                        )
                        return cc

                    lax.fori_loop(0, _G // _LN, xform, 0)
                for g in range(_RG):
                    pltpu.async_copy(
                        table.at[idxv.at[b].at[g]],
                        rows.at[b].at[pl.ds(g * _G, _G)],
                        sems[b],
                    )

            def consume(b):
                for g in range(_RG):
                    pltpu.make_async_copy(
                        table.at[idxv.at[b].at[g]],
                        rows.at[b].at[pl.ds(g * _G, _G)],
                        sems[b],
                    ).wait()

                    def scale(l, cc):
                        gain = av[b, g, pl.ds(l * _LN, _LN)]
                        for lane in range(_LN):
                            e = g * _G + l * _LN + lane
                            bc = _bcast(gain, lane)
                            rows[b, e, pl.ds(0, _LN)] = (
                                rows[b, e, pl.ds(0, _LN)] * bc
                            )
                        return cc

                    lax.fori_loop(0, _G // _LN, scale, 0)

                    pltpu.async_copy(
                        rows.at[b].at[pl.ds(g * _G, _G)],
                        acc.at[dstv.at[b].at[g]],
                        ssems[b],
                        add=True,
                    )

                for g in range(_RG):
                    pltpu.make_async_copy(
                        rows.at[b].at[pl.ds(g * _G, _G)],
                        acc.at[dstv.at[b].at[g]],
                        ssems[b],
                    ).wait()

            prefetch(0, 0)

            def pair(j, carry):
                prefetch(1, 2 * j + 1)
                consume(0)
                prefetch(0, 2 * j + 2)
                consume(1)
                return carry

            lax.fori_loop(0, n_pairs - 1, pair, 0)
            prefetch(1, n_chunks - 1)
            consume(0)
            consume(1)

            plsc.subcore_barrier()
            pltpu.sync_copy(
                acc.at[pl.ds(base_row, rows_per_tile)],
                out.at[pl.ds(base_row, rows_per_tile), pl.ds(sid * _DH, _DH)],
            )

    return side_kernel


@functools.lru_cache(maxsize=None)
def _make_dense_kernel(n_nodes, d_in, d_out, br):
    assert n_nodes % br == 0 and d_in % _DH == 0

    def body(e_ref, side_ref, wg_ref, bg_ref, wb_ref, bb_ref, e1_ref, n1_ref):
        e = e_ref[...]
        side = side_ref[...]
        z = jnp.dot(e + side, wg_ref[...], preferred_element_type=jnp.float32)
        z = z + bg_ref[...]
        sum_emb = jnp.where(z >= 0, z, 0.01 * z)
        z2 = jnp.dot(e * side, wb_ref[...], preferred_element_type=jnp.float32)
        z2 = z2 + bb_ref[...]
        bi_emb = jnp.where(z2 >= 0, z2, 0.01 * z2)
        outv = sum_emb + bi_emb
        e1_ref[...] = outv
        nrm = jnp.sqrt(jnp.sum(outv * outv, axis=1, keepdims=True))
        n1_ref[...] = outv / jnp.maximum(nrm, 1e-12)

    return pl.pallas_call(
        body,
        grid=(n_nodes // br,),
        in_specs=[
            pl.BlockSpec((br, d_in), lambda i: (i, 0)),
            pl.BlockSpec((br, d_in), lambda i: (i, 0)),
            pl.BlockSpec((d_in, d_out), lambda i: (0, 0)),
            pl.BlockSpec((1, d_out), lambda i: (0, 0)),
            pl.BlockSpec((d_in, d_out), lambda i: (0, 0)),
            pl.BlockSpec((1, d_out), lambda i: (0, 0)),
        ],
        out_specs=[
            pl.BlockSpec((br, d_out), lambda i: (i, 0)),
            pl.BlockSpec((br, d_out), lambda i: (i, 0)),
        ],
        out_shape=[
            jax.ShapeDtypeStruct((n_nodes, d_out), jnp.float32),
            jax.ShapeDtypeStruct((n_nodes, d_out), jnp.float32),
        ],
    )


def kernel(ego_embeddings, edge_index, a_values,
           W_gc0, b_gc0, W_bi0, b_bi0,
           W_gc1, b_gc1, W_bi1, b_bi1):
    n, emb = ego_embeddings.shape
    n_edges = a_values.shape[0]
    h1 = W_gc0.shape[1]
    h2 = W_gc1.shape[1]

    q = _NS * _CH * 2
    e_pad = ((n_edges + q - 1) // q) * q
    pad = e_pad - n_edges
    src = edge_index[0]
    dst = edge_index[1]
    a = a_values
    if pad:
        pad_idx = (jnp.arange(pad, dtype=jnp.int32) * 97) % n
        src = jnp.concatenate([src, pad_idx])
        dst = jnp.concatenate([dst, pad_idx])
        a = jnp.concatenate([a, jnp.zeros((pad,), jnp.float32)])
    srcg = src.reshape(e_pad // _G, _G)
    dstg = dst.reshape(e_pad // _G, _G)
    ag = a.reshape(e_pad // _G, _G)

    n_pad = ((n + _NS * 8 - 1) // (_NS * 8)) * (_NS * 8)

    ns0 = emb // _DH
    table0 = ego_embeddings.reshape(ns0 * n, _DH)
    side0 = _make_side_kernel(n_pad, e_pad, ns0)(table0, srcg, dstg, ag)
    e1, n1 = _make_dense_kernel(n, emb, h1, 2000)(
        ego_embeddings, side0, W_gc0, b_gc0, W_bi0, b_bi0
    )

    ns1 = h1 // _DH
    table1 = e1.reshape(ns1 * n, _DH)
    side1 = _make_side_kernel(n_pad, e_pad, ns1)(table1, srcg, dstg, ag)
    _, n2 = _make_dense_kernel(n, h1, h2, 2000)(
        e1, side1, W_gc1, b_gc1, W_bi1, b_bi1
    )

    return jnp.concatenate([ego_embeddings, n1, n2], axis=1)
